# Initial kernel scaffold; baseline (speedup 1.0000x reference)
import jax
import jax.numpy as jnp
from jax.experimental import pallas as pl


def kernel(features_0, graph_ids):
    raise NotImplementedError("write your pallas kernel here")


# speedup vs baseline: 2.3527x; 2.3527x over previous
import functools

import jax
import jax.numpy as jnp
from jax import lax
from jax.experimental import pallas as pl
from jax.experimental.pallas import tpu as pltpu
from jax.experimental.pallas import tpu_sc as plsc

N_NODES = 100000
D_FEAT = 128
NUM_GRAPHS = 64
NUM_WORKERS = 32
ROWS_PER_WORKER = 3136
CHUNK = 224
NUM_CHUNKS = ROWS_PER_WORKER // CHUNK
NLANES = 16
NVEC = D_FEAT // NLANES


def _sc_partials(feats, ids):
    mesh = plsc.VectorSubcoreMesh(core_axis_name="c", subcore_axis_name="s")

    @functools.partial(
        pl.kernel,
        out_type=jax.ShapeDtypeStruct((NUM_WORKERS, NUM_GRAPHS, D_FEAT),
                                      jnp.float32),
        mesh=mesh,
        scratch_types=[
            pltpu.VMEM((CHUNK, D_FEAT), jnp.float32),
            pltpu.VMEM((CHUNK,), jnp.int32),
            pltpu.VMEM((NUM_GRAPHS, D_FEAT), jnp.float32),
        ],
    )
    def k(feat_hbm, ids_hbm, out_hbm, fbuf, ibuf, tab):
        wid = lax.axis_index("s") * 2 + lax.axis_index("c")
        base = jnp.minimum(wid * ROWS_PER_WORKER, N_NODES - ROWS_PER_WORKER)

        neg = jnp.full((NLANES,), -jnp.inf, dtype=jnp.float32)

        def init_body(g, _):
            for j in range(NVEC):
                tab[g, pl.ds(j * NLANES, NLANES)] = neg
            return 0

        lax.fori_loop(0, NUM_GRAPHS, init_body, 0)

        def chunk_body(c, _):
            off = base + c * CHUNK
            pltpu.sync_copy(feat_hbm.at[pl.ds(off, CHUNK), :], fbuf)
            pltpu.sync_copy(ids_hbm.at[pl.ds(off, CHUNK)], ibuf)

            def grp_body(q, _):
                r0 = q * NLANES
                idvec = ibuf[pl.ds(r0, NLANES)]
                for i in range(NLANES):
                    g = idvec[i]
                    for j in range(NVEC):
                        sl = pl.ds(j * NLANES, NLANES)
                        tab[g, sl] = jnp.maximum(tab[g, sl], fbuf[r0 + i, sl])
                return 0

            lax.fori_loop(0, CHUNK // NLANES, grp_body, 0)
            return 0

        lax.fori_loop(0, NUM_CHUNKS, chunk_body, 0)
        pltpu.sync_copy(tab, out_hbm.at[wid])

    return k(feats, ids)


def _reduce_body(p_ref, o_ref):
    o_ref[...] = jnp.max(p_ref[...], axis=0)


def kernel(features_0, graph_ids):
    feats = features_0.reshape(N_NODES, D_FEAT)
    ids = graph_ids.astype(jnp.int32)
    partials = _sc_partials(feats, ids)
    out = pl.pallas_call(
        _reduce_body,
        out_shape=jax.ShapeDtypeStruct((NUM_GRAPHS, D_FEAT), jnp.float32),
    )(partials)
    return out


# speedup vs baseline: 6.7186x; 2.8557x over previous
import functools

import jax
import jax.numpy as jnp
from jax import lax
from jax.experimental import pallas as pl
from jax.experimental.pallas import tpu as pltpu
from jax.experimental.pallas import tpu_sc as plsc

N_NODES = 100000
D_FEAT = 128
NUM_GRAPHS = 64
NUM_WORKERS = 32
ROWS_PER_WORKER = 3136
CHUNK = 224
NUM_CHUNKS = ROWS_PER_WORKER // CHUNK
NLANES = 16
NVEC = D_FEAT // NLANES


def _sc_partials(feats, ids):
    mesh = plsc.VectorSubcoreMesh(core_axis_name="c", subcore_axis_name="s")

    @functools.partial(
        pl.kernel,
        out_type=jax.ShapeDtypeStruct((NUM_WORKERS, NUM_GRAPHS, D_FEAT),
                                      jnp.float32),
        mesh=mesh,
        scratch_types=[
            pltpu.VMEM((CHUNK, D_FEAT), jnp.float32),
            pltpu.VMEM((CHUNK, D_FEAT), jnp.float32),
            pltpu.VMEM((CHUNK,), jnp.int32),
            pltpu.VMEM((CHUNK,), jnp.int32),
            pltpu.VMEM((NUM_GRAPHS, D_FEAT), jnp.float32),
            pltpu.SemaphoreType.DMA,
            pltpu.SemaphoreType.DMA,
            pltpu.SemaphoreType.DMA,
            pltpu.SemaphoreType.DMA,
        ],
    )
    def k(feat_hbm, ids_hbm, out_hbm, fb0, fb1, ib0, ib1, tab,
          fs0, fs1, is0, is1):
        wid = lax.axis_index("s") * 2 + lax.axis_index("c")
        base = jnp.minimum(wid * ROWS_PER_WORKER, N_NODES - ROWS_PER_WORKER)
        fbuf = (fb0, fb1)
        ibuf = (ib0, ib1)
        fsem = (fs0, fs1)
        isem = (is0, is1)

        def start(c, b):
            off = base + c * CHUNK
            pltpu.async_copy(feat_hbm.at[pl.ds(off, CHUNK), :], fbuf[b],
                             fsem[b])
            pltpu.async_copy(ids_hbm.at[pl.ds(off, CHUNK)], ibuf[b],
                             isem[b])

        def wait(b):
            pltpu.make_async_copy(feat_hbm.at[pl.ds(0, CHUNK), :], fbuf[b],
                                  fsem[b]).wait()
            pltpu.make_async_copy(ids_hbm.at[pl.ds(0, CHUNK)], ibuf[b],
                                  isem[b]).wait()

        start(0, 0)
        start(1, 1)

        neg = jnp.full((NLANES,), -jnp.inf, dtype=jnp.float32)

        def init_body(g, _):
            for j in range(NVEC):
                tab[g, pl.ds(j * NLANES, NLANES)] = neg
            return 0

        lax.fori_loop(0, NUM_GRAPHS, init_body, 0)

        def compute(b):
            def grp_body(q, _):
                r0 = q * NLANES
                idvec = ibuf[b][pl.ds(r0, NLANES)]
                g0 = idvec[0]

                def fast(_):
                    for j in range(NVEC):
                        sl = pl.ds(j * NLANES, NLANES)
                        m = [fbuf[b][r0 + i, sl] for i in range(NLANES)]
                        while len(m) > 1:
                            m = ([jnp.maximum(m[2 * t], m[2 * t + 1])
                                  for t in range(len(m) // 2)]
                                 + m[len(m) // 2 * 2:])
                        tab[g0, sl] = jnp.maximum(tab[g0, sl], m[0])
                    return 0

                def slow(_):
                    for i in range(NLANES):
                        g = idvec[i]
                        for j in range(NVEC):
                            sl = pl.ds(j * NLANES, NLANES)
                            tab[g, sl] = jnp.maximum(tab[g, sl],
                                                     fbuf[b][r0 + i, sl])
                    return 0

                lax.cond(g0 == idvec[NLANES - 1], fast, slow, 0)
                return 0

            lax.fori_loop(0, CHUNK // NLANES, grp_body, 0)

        def body2(cc, _):
            for b in range(2):
                c = 2 * cc + b
                wait(b)
                compute(b)

                @pl.when(c + 2 < NUM_CHUNKS)
                def _():
                    start(c + 2, b)
            return 0

        lax.fori_loop(0, NUM_CHUNKS // 2, body2, 0)
        pltpu.sync_copy(tab, out_hbm.at[wid])

    return k(feats, ids)


def _reduce_body(p_ref, o_ref):
    o_ref[...] = jnp.max(p_ref[...], axis=0)


def kernel(features_0, graph_ids):
    feats = features_0.reshape(N_NODES, D_FEAT)
    ids = graph_ids.astype(jnp.int32)
    partials = _sc_partials(feats, ids)
    out = pl.pallas_call(
        _reduce_body,
        out_shape=jax.ShapeDtypeStruct((NUM_GRAPHS, D_FEAT), jnp.float32),
    )(partials)
    return out
